# Initial kernel scaffold; baseline (speedup 1.0000x reference)
import jax
import jax.numpy as jnp
from jax.experimental import pallas as pl


def kernel(scores, bbox_deltas, im_info):
    raise NotImplementedError("write your pallas kernel here")


# speedup vs baseline: 99.9883x; 99.9883x over previous
import numpy as np
import jax
import jax.numpy as jnp
from jax import lax
from jax.experimental import pallas as pl
from jax.experimental.pallas import tpu as pltpu

_FEAT_STRIDE = 16
_PRE = 12000
_POST = 2000
_THR = 0.7
_S = 128
_NB = 94
_NPAD = _S * _NB


def _whctrs(a):
    w = a[2] - a[0] + 1.0
    h = a[3] - a[1] + 1.0
    return w, h, a[0] + 0.5 * (w - 1), a[1] + 0.5 * (h - 1)


def _mkanchors(ws, hs, xc, yc):
    ws = np.asarray(ws, dtype=np.float64).reshape(-1, 1)
    hs = np.asarray(hs, dtype=np.float64).reshape(-1, 1)
    return np.hstack((xc - 0.5 * (ws - 1), yc - 0.5 * (hs - 1),
                      xc + 0.5 * (ws - 1), yc + 0.5 * (hs - 1)))


def _gen_anchors():
    ratios = np.array([0.5, 1.0, 2.0])
    scales = np.array([8, 16, 32])
    base = np.array([1.0, 1.0, 16.0, 16.0]) - 1
    w, h, xc, yc = _whctrs(base)
    size = w * h
    ws = np.round(np.sqrt(size / ratios))
    hs = np.round(ws * ratios)
    ra = _mkanchors(ws, hs, xc, yc)
    rows = []
    for i in range(ra.shape[0]):
        w, h, xc, yc = _whctrs(ra[i])
        rows.append(_mkanchors(w * scales, h * scales, xc, yc))
    return np.vstack(rows).astype(np.float32)


def _anchor_grid(H, W):
    a0 = _gen_anchors()
    A = a0.shape[0]
    shift_x = np.arange(W) * _FEAT_STRIDE
    shift_y = np.arange(H) * _FEAT_STRIDE
    sx, sy = np.meshgrid(shift_x, shift_y)
    shifts = np.vstack((sx.ravel(), sy.ravel(), sx.ravel(), sy.ravel()))
    shifts = shifts.transpose().astype(np.float32)
    K = shifts.shape[0]
    anchors = a0[None, :, :] + shifts[:, None, :]
    return anchors.reshape(1, K * A, 4), A


def _nms_body(x1_ref, y1_ref, x2_ref, y2_ref, col_ref, keptm_ref,
              alive_ref, kc_ref):
    f32 = jnp.float32
    eye = (lax.broadcasted_iota(jnp.int32, (_S, _S), 0)
           == lax.broadcasted_iota(jnp.int32, (_S, _S), 1)).astype(f32)
    tri = (lax.broadcasted_iota(jnp.int32, (_S, _S), 0)
           < lax.broadcasted_iota(jnp.int32, (_S, _S), 1))
    ones_row = jnp.ones((1, _S), f32)
    gidx = (lax.broadcasted_iota(jnp.int32, (_NB, _S), 0) * _S
            + lax.broadcasted_iota(jnp.int32, (_NB, _S), 1))

    kc_ref[0] = 0
    alive_ref[:, :] = (gidx < _PRE).astype(f32)
    keptm_ref[0, :, :] = jnp.zeros((_NB, _S), f32)

    def block_body(b, carry):
        @pl.when(kc_ref[0] < _POST)
        def _():
            kx1 = col_ref[0, pl.ds(b * _S, _S), 0:1]
            ky1 = col_ref[0, pl.ds(b * _S, _S), 1:2]
            kx2 = col_ref[0, pl.ds(b * _S, _S), 2:3]
            ky2 = col_ref[0, pl.ds(b * _S, _S), 3:4]
            karea = (kx2 - kx1 + 1.0) * (ky2 - ky1 + 1.0)
            cx1 = x1_ref[0, pl.ds(b, 1), :]
            cy1 = y1_ref[0, pl.ds(b, 1), :]
            cx2 = x2_ref[0, pl.ds(b, 1), :]
            cy2 = y2_ref[0, pl.ds(b, 1), :]
            carea = (cx2 - cx1 + 1.0) * (cy2 - cy1 + 1.0)

            xx1 = jnp.maximum(kx1, cx1)
            yy1 = jnp.maximum(ky1, cy1)
            xx2 = jnp.minimum(kx2, cx2)
            yy2 = jnp.minimum(ky2, cy2)
            w = jnp.maximum(0.0, xx2 - xx1 + 1.0)
            h = jnp.maximum(0.0, yy2 - yy1 + 1.0)
            inter = w * h
            ovr = inter / (karea + carea - inter)
            mtri = ((ovr > _THR) & tri).astype(f32)

            alive = alive_ref[pl.ds(b, 1), :]

            def fix_cond(st):
                return st[1]

            def fix_body(st):
                keep, _ = st
                sup = lax.dot_general(keep, mtri, (((1,), (0,)), ((), ())),
                                      preferred_element_type=f32)
                nk = alive * (sup < 0.5).astype(f32)
                return nk, jnp.any(nk != keep)

            keep, _ = lax.while_loop(fix_cond, fix_body,
                                     (alive, jnp.bool_(True)))

            keptm_ref[0, pl.ds(b, 1), :] = keep
            kc_ref[0] = kc_ref[0] + jnp.sum(keep).astype(jnp.int32)

            kcol = lax.dot_general(eye, keep, (((1,), (1,)), ((), ())),
                                   preferred_element_type=f32) > 0.5

            def row_body(r, c2):
                rx1 = x1_ref[0, pl.ds(r, 1), :]
                ry1 = y1_ref[0, pl.ds(r, 1), :]
                rx2 = x2_ref[0, pl.ds(r, 1), :]
                ry2 = y2_ref[0, pl.ds(r, 1), :]
                rarea = (rx2 - rx1 + 1.0) * (ry2 - ry1 + 1.0)
                a1 = jnp.maximum(kx1, rx1)
                b1 = jnp.maximum(ky1, ry1)
                a2 = jnp.minimum(kx2, rx2)
                b2 = jnp.minimum(ky2, ry2)
                ww = jnp.maximum(0.0, a2 - a1 + 1.0)
                hh = jnp.maximum(0.0, b2 - b1 + 1.0)
                it = ww * hh
                ov = it / (karea + rarea - it)
                supm = ((ov > _THR) & kcol).astype(f32)
                cnt = lax.dot_general(ones_row, supm, (((1,), (0,)), ((), ())),
                                      preferred_element_type=f32)
                alive_ref[pl.ds(r, 1), :] = (alive_ref[pl.ds(r, 1), :]
                                             * (cnt < 0.5).astype(f32))
                return c2

            lax.fori_loop(b + 1, _NB, row_body, 0)

        return carry

    lax.fori_loop(0, _NB, block_body, 0)


def _run_nms(x1r, y1r, x2r, y2r, colp):
    B = x1r.shape[0]
    spec_row = pl.BlockSpec((1, _NB, _S), lambda i: (i, 0, 0))
    spec_col = pl.BlockSpec((1, _NPAD, 4), lambda i: (i, 0, 0))
    return pl.pallas_call(
        _nms_body,
        grid=(B,),
        in_specs=[spec_row, spec_row, spec_row, spec_row, spec_col],
        out_specs=spec_row,
        out_shape=jax.ShapeDtypeStruct((B, _NB, _S), jnp.float32),
        scratch_shapes=[pltpu.VMEM((_NB, _S), jnp.float32),
                        pltpu.SMEM((1,), jnp.int32)],
    )(x1r, y1r, x2r, y2r, colp)


def kernel(scores, bbox_deltas, im_info):
    B = scores.shape[0]
    H, W = scores.shape[2], scores.shape[3]
    anchors_np, A = _anchor_grid(H, W)
    anchors = jnp.asarray(anchors_np)
    anchors = jnp.broadcast_to(anchors, (B, anchors.shape[1], 4))

    sc = scores[:, A:, :, :]
    sc_flat = jnp.transpose(sc, (0, 2, 3, 1)).reshape(B, -1)
    deltas = jnp.transpose(bbox_deltas, (0, 2, 3, 1)).reshape(B, -1, 4)

    widths = anchors[:, :, 2] - anchors[:, :, 0] + 1.0
    heights = anchors[:, :, 3] - anchors[:, :, 1] + 1.0
    ctr_x = anchors[:, :, 0] + 0.5 * widths
    ctr_y = anchors[:, :, 1] + 0.5 * heights
    dx = deltas[:, :, 0]
    dy = deltas[:, :, 1]
    dw = deltas[:, :, 2]
    dh = deltas[:, :, 3]
    pcx = dx * widths + ctr_x
    pcy = dy * heights + ctr_y
    pw = jnp.exp(dw) * widths
    ph = jnp.exp(dh) * heights
    proposals = jnp.stack([pcx - 0.5 * pw, pcy - 0.5 * ph,
                           pcx + 0.5 * pw, pcy + 0.5 * ph], axis=2)
    hh = im_info[:, 0:1]
    ww = im_info[:, 1:2]
    x1 = jnp.clip(proposals[:, :, 0], 0.0, ww - 1.0)
    y1 = jnp.clip(proposals[:, :, 1], 0.0, hh - 1.0)
    x2 = jnp.clip(proposals[:, :, 2], 0.0, ww - 1.0)
    y2 = jnp.clip(proposals[:, :, 3], 0.0, hh - 1.0)
    proposals = jnp.stack([x1, y1, x2, y2], axis=2)

    order = jnp.argsort(-sc_flat, axis=1)[:, :_PRE]
    props = jnp.take_along_axis(proposals, order[:, :, None], axis=1)
    propsp = jnp.pad(props, ((0, 0), (0, _NPAD - _PRE), (0, 0)))

    x1r = propsp[:, :, 0].reshape(B, _NB, _S)
    y1r = propsp[:, :, 1].reshape(B, _NB, _S)
    x2r = propsp[:, :, 2].reshape(B, _NB, _S)
    y2r = propsp[:, :, 3].reshape(B, _NB, _S)

    keptm = _run_nms(x1r, y1r, x2r, y2r, propsp)

    flat = keptm.reshape(B, _NPAD) > 0.5
    pos = jnp.cumsum(flat.astype(jnp.int32), axis=1) - 1
    posc = jnp.where(flat & (pos < _POST), pos, _POST)
    out4 = jax.vmap(
        lambda p, bx: jnp.zeros((_POST + 1, 4), jnp.float32).at[p].add(bx)
    )(posc, propsp)
    out4 = out4[:, :_POST]
    col0 = jnp.broadcast_to(
        jnp.arange(B, dtype=jnp.float32)[:, None, None], (B, _POST, 1))
    return jnp.concatenate([col0, out4], axis=2)
